# Initial kernel scaffold; baseline (speedup 1.0000x reference)
import jax
import jax.numpy as jnp
from jax.experimental import pallas as pl


def kernel(hidden_states, gate_w, gate_b, w1, b1, w2, b2):
    raise NotImplementedError("write your pallas kernel here")


# speedup vs baseline: 11.3691x; 11.3691x over previous
import functools

import jax
import jax.numpy as jnp
from jax.experimental import pallas as pl
from jax.experimental.pallas import tpu as pltpu

_E = 64
_TOPK = 2
_H = 768
_I = 768
_T = 2048
_R = _T * _TOPK
_BM = 64
_MAXPAD = _R + _E * _BM
_NBLK = _MAXPAD // _BM

_INTERPRET = False


def _ffn_block(info_ref, x_ref, w1_ref, b1_ref, sel_ref, w2_ref, b2_ref,
               wp_ref, out_ref):
    i = pl.program_id(0)
    nused = info_ref[_NBLK]

    @pl.when(i < nused)
    def _():
        x = x_ref[...]
        w1e = w1_ref[0]
        h = jax.lax.dot_general(x, w1e, (((1,), (1,)), ((), ())),
                                preferred_element_type=jnp.float32)
        h = h + b1_ref[0, 0]
        hr = pltpu.roll(h, 2 * _I - 1, 1)
        p = h * jax.nn.sigmoid(1.702 * h) * (hr + 1.0)
        s = jax.lax.dot_general(p, sel_ref[...], (((1,), (0,)), ((), ())),
                                preferred_element_type=jnp.float32)
        y = jax.lax.dot_general(s, w2_ref[0], (((1,), (1,)), ((), ())),
                                preferred_element_type=jnp.float32)
        y = y + b2_ref[0, 0]
        out_ref[...] = y * wp_ref[0, 0][:, None]


def kernel(hidden_states, gate_w, gate_b, w1, b1, w2, b2):
    bsz, seq, hd = hidden_states.shape
    x2 = hidden_states.reshape(-1, hd)

    logits = x2 @ gate_w.T + gate_b
    probs = jax.nn.softmax(logits, axis=-1)
    vals, idx = jax.lax.top_k(probs, _TOPK)
    vals = vals / jnp.sum(vals, axis=-1, keepdims=True)

    e_flat = idx.reshape(-1).astype(jnp.int32)
    v_flat = vals.reshape(-1)
    order = jnp.argsort(e_flat, stable=True)
    rank = jnp.zeros((_R,), jnp.int32).at[order].set(
        jnp.arange(_R, dtype=jnp.int32))
    counts = jnp.bincount(e_flat, length=_E).astype(jnp.int32)
    pcounts = ((counts + _BM - 1) // _BM) * _BM
    pc_cum = jnp.cumsum(pcounts).astype(jnp.int32)
    pstart = pc_cum - pcounts
    g_cum = jnp.cumsum(counts).astype(jnp.int32)
    gstart = g_cum - counts
    total_pad = pc_cum[-1]
    nused = (total_pad // _BM).astype(jnp.int32)
    pos = pstart[e_flat] + (rank - gstart[e_flat])
    src_tok = jnp.zeros((_MAXPAD,), jnp.int32).at[pos].set(
        jnp.arange(_R, dtype=jnp.int32) // _TOPK)
    w_pad = jnp.zeros((_MAXPAD,), jnp.float32).at[pos].set(v_flat)
    queries = (jnp.arange(_NBLK, dtype=jnp.int32) * _BM).astype(jnp.int32)
    be = jnp.searchsorted(pc_cum, queries, side="right").astype(jnp.int32)
    be_last = be[jnp.maximum(nused - 1, 0)]
    be = jnp.where(queries < total_pad, be, be_last)
    info = jnp.concatenate([be, nused[None]])

    x_pad = x2[src_tok]

    b1r = b1.reshape(_E, 1, 2 * _I)
    b2r = b2.reshape(_E, 1, _H)
    wpr = w_pad.reshape(_NBLK, 1, _BM)
    sel = (jnp.arange(2 * _I, dtype=jnp.int32)[:, None]
           == 2 * jnp.arange(_I, dtype=jnp.int32)[None, :]).astype(jnp.float32)
    grid_spec = pltpu.PrefetchScalarGridSpec(
        num_scalar_prefetch=1,
        grid=(_NBLK,),
        in_specs=[
            pl.BlockSpec((_BM, _H), lambda i, info: (i, 0)),
            pl.BlockSpec((1, 2 * _I, _H), lambda i, info: (info[i], 0, 0)),
            pl.BlockSpec((1, 1, 2 * _I), lambda i, info: (info[i], 0, 0)),
            pl.BlockSpec((2 * _I, _I), lambda i, info: (0, 0)),
            pl.BlockSpec((1, _H, _I), lambda i, info: (info[i], 0, 0)),
            pl.BlockSpec((1, 1, _H), lambda i, info: (info[i], 0, 0)),
            pl.BlockSpec((1, 1, _BM), lambda i, info: (i, 0, 0)),
        ],
        out_specs=pl.BlockSpec((_BM, _H), lambda i, info: (i, 0)),
    )
    y_pad = pl.pallas_call(
        _ffn_block,
        grid_spec=grid_spec,
        out_shape=jax.ShapeDtypeStruct((_MAXPAD, _H), jnp.float32),
        compiler_params=pltpu.CompilerParams(
            dimension_semantics=("arbitrary",)),
        interpret=_INTERPRET,
    )(info, x_pad, w1, b1r, sel, w2, b2r, wpr)

    p2 = pos.reshape(_T, _TOPK)
    out2 = y_pad[p2[:, 0]] + y_pad[p2[:, 1]]
    return out2.reshape(bsz, seq, hd)
